# Initial kernel scaffold; baseline (speedup 1.0000x reference)
import functools

import jax
import jax.numpy as jnp
from jax import lax
from jax.experimental import pallas as pl
from jax.experimental.pallas import tpu as pltpu
from jax.experimental.pallas import tpu_sc as plsc

_NUM_NODES = 50000
_EMBED_DIM = 64
_NUM_EDGES = 800000

_B = 2 * _NUM_EDGES
_NC = 2
_NS = 16
_NW = _NC * _NS
_PER_W = _B // _NW
_CHUNK = 80
_NCHUNK = _PER_W // _CHUNK


def _gather_body(idx_hbm, table_hbm, out_hbm, idx_v, rows_v, sem):
    wid = lax.axis_index("s") * _NC + lax.axis_index("c")
    base = wid * _PER_W

    def step(j, carry):
        off = base + j * _CHUNK
        pltpu.sync_copy(idx_hbm.at[pl.ds(off, _CHUNK)], idx_v)
        pltpu.async_copy(table_hbm.at[idx_v], rows_v, sem).wait()
        pltpu.sync_copy(rows_v, out_hbm.at[pl.ds(off, _CHUNK)])
        return carry

    lax.fori_loop(0, _NCHUNK, step, 0)


@jax.jit
def _gather(idx, table):
    mesh = plsc.VectorSubcoreMesh(core_axis_name="c", subcore_axis_name="s")
    return pl.kernel(
        _gather_body,
        out_type=jax.ShapeDtypeStruct((_B, _EMBED_DIM), jnp.float32),
        mesh=mesh,
        scratch_types=[
            pltpu.VMEM((_CHUNK,), jnp.int32),
            pltpu.VMEM((_CHUNK, _EMBED_DIM), jnp.float32),
            pltpu.SemaphoreType.DMA,
        ],
    )(idx, table)


def kernel(edge_index, embedding_weight):
    idx = edge_index.reshape(-1).astype(jnp.int32)
    out = _gather(idx, embedding_weight)
    return out.reshape(2, _NUM_EDGES, _EMBED_DIM)


# speedup vs baseline: 2.0386x; 2.0386x over previous
import functools

import jax
import jax.numpy as jnp
from jax import lax
from jax.experimental import pallas as pl
from jax.experimental.pallas import tpu as pltpu
from jax.experimental.pallas import tpu_sc as plsc

_NUM_NODES = 50000
_EMBED_DIM = 64
_NUM_EDGES = 800000

_B = 2 * _NUM_EDGES
_NC = 2
_NS = 16
_NW = _NC * _NS
_PER_W = _B // _NW
_CHUNK = 80
_NCHUNK = _PER_W // _CHUNK


def _gather_body(idx_hbm, table_hbm, out_hbm, idx_v, rows_v, sem):
    wid = lax.axis_index("s") * _NC + lax.axis_index("c")
    base = wid * _PER_W

    def step(j, carry):
        off = base + j * _CHUNK
        pltpu.sync_copy(idx_hbm.at[pl.ds(off, _CHUNK)], idx_v)
        pltpu.async_copy(table_hbm.at[idx_v], rows_v, sem).wait()
        pltpu.sync_copy(rows_v, out_hbm.at[pl.ds(off, _CHUNK)])
        return carry

    lax.fori_loop(0, _NCHUNK, step, 0)


@jax.jit
def _gather(idx, table):
    mesh = plsc.VectorSubcoreMesh(core_axis_name="c", subcore_axis_name="s")
    return pl.kernel(
        _gather_body,
        out_type=jax.ShapeDtypeStruct((_B, _EMBED_DIM), jnp.float32),
        mesh=mesh,
        scratch_types=[
            pltpu.VMEM((_CHUNK,), jnp.int32),
            pltpu.VMEM((_CHUNK, _EMBED_DIM), jnp.float32),
            pltpu.SemaphoreType.DMA,
        ],
        compiler_params=pltpu.CompilerParams(use_tc_tiling_on_sc=False),
    )(idx, table)


def kernel(edge_index, embedding_weight):
    idx = edge_index.reshape(-1).astype(jnp.int32)
    out = _gather(idx, embedding_weight)
    return out.reshape(2, _NUM_EDGES, _EMBED_DIM)


# speedup vs baseline: 3.2342x; 1.5864x over previous
import jax
import jax.numpy as jnp
from jax import lax
from jax.experimental import pallas as pl
from jax.experimental.pallas import tpu as pltpu
from jax.experimental.pallas import tpu_sc as plsc

_NUM_NODES = 50000
_EMBED_DIM = 64
_NUM_EDGES = 800000

_B = 2 * _NUM_EDGES
_NC = 2
_NS = 16
_NW = _NC * _NS
_PER_W = _B // _NW
_CHUNK = 80
_NCHUNK = _PER_W // _CHUNK
_G = 5
_GROWS = _G * _CHUNK
_NGROUP = _NCHUNK // _G


def _gather_body(idx_hbm, table_hbm, out_hbm, idx_v, rows_v, gsem, wsem):
    wid = lax.axis_index("s") * _NC + lax.axis_index("c")
    base = wid * _PER_W

    pltpu.sync_copy(idx_hbm.at[wid], idx_v)

    def fire_group(g, half):
        for i in range(_G):
            chunk = g * _G + i
            pltpu.async_copy(
                table_hbm.at[idx_v.at[chunk]],
                rows_v.at[pl.ds(half * _GROWS + i * _CHUNK, _CHUNK)],
                gsem.at[half],
            )

    def drain_gathers(half):
        pltpu.make_async_copy(
            table_hbm.at[pl.ds(0, _GROWS)],
            rows_v.at[pl.ds(half * _GROWS, _GROWS)],
            gsem.at[half],
        ).wait()

    def fire_writeback(g, half):
        pltpu.async_copy(
            rows_v.at[pl.ds(half * _GROWS, _GROWS)],
            out_hbm.at[pl.ds(base + g * _GROWS, _GROWS)],
            wsem.at[half],
        )

    def drain_writeback(half):
        pltpu.make_async_copy(
            table_hbm.at[pl.ds(0, _GROWS)],
            rows_v.at[pl.ds(half * _GROWS, _GROWS)],
            wsem.at[half],
        ).wait()

    fire_group(0, 0)

    def step(g, carry):
        h = lax.rem(g, 2)
        hn = 1 - h

        @pl.when(g + 1 < _NGROUP)
        def _():
            @pl.when(g >= 1)
            def _():
                drain_writeback(hn)

            fire_group(g + 1, hn)

        drain_gathers(h)
        fire_writeback(g, h)
        return carry

    lax.fori_loop(0, _NGROUP, step, 0)

    drain_writeback((_NGROUP - 1) % 2)
    drain_writeback(_NGROUP % 2)


@jax.jit
def _gather(idx, table):
    mesh = plsc.VectorSubcoreMesh(core_axis_name="c", subcore_axis_name="s")
    return pl.kernel(
        _gather_body,
        out_type=jax.ShapeDtypeStruct((_B, _EMBED_DIM), jnp.float32),
        mesh=mesh,
        scratch_types=[
            pltpu.VMEM((_NCHUNK, _CHUNK), jnp.int32),
            pltpu.VMEM((2 * _GROWS, _EMBED_DIM), jnp.float32),
            pltpu.SemaphoreType.DMA((2,)),
            pltpu.SemaphoreType.DMA((2,)),
        ],
        compiler_params=pltpu.CompilerParams(use_tc_tiling_on_sc=False),
    )(idx, table)


def kernel(edge_index, embedding_weight):
    idx = edge_index.reshape(_NW, _NCHUNK, _CHUNK).astype(jnp.int32)
    out = _gather(idx, embedding_weight)
    return out.reshape(2, _NUM_EDGES, _EMBED_DIM)


# speedup vs baseline: 3.2446x; 1.0032x over previous
import jax
import jax.numpy as jnp
from jax import lax
from jax.experimental import pallas as pl
from jax.experimental.pallas import tpu as pltpu
from jax.experimental.pallas import tpu_sc as plsc

_NUM_NODES = 50000
_EMBED_DIM = 64
_NUM_EDGES = 800000

_B = 2 * _NUM_EDGES
_NC = 2
_NS = 16
_NW = _NC * _NS
_PER_W = _B // _NW
_CHUNK = 400
_NCHUNK = _PER_W // _CHUNK
_G = 1
_GROWS = _G * _CHUNK
_NGROUP = _NCHUNK // _G


def _gather_body(idx_hbm, table_hbm, out_hbm, idx_v, rows_v, gsem, wsem):
    wid = lax.axis_index("s") * _NC + lax.axis_index("c")
    base = wid * _PER_W

    pltpu.sync_copy(idx_hbm.at[wid], idx_v)

    def fire_group(g, half):
        for i in range(_G):
            chunk = g * _G + i
            pltpu.async_copy(
                table_hbm.at[idx_v.at[chunk]],
                rows_v.at[pl.ds(half * _GROWS + i * _CHUNK, _CHUNK)],
                gsem.at[half],
            )

    def drain_gathers(half):
        pltpu.make_async_copy(
            table_hbm.at[pl.ds(0, _GROWS)],
            rows_v.at[pl.ds(half * _GROWS, _GROWS)],
            gsem.at[half],
        ).wait()

    def fire_writeback(g, half):
        pltpu.async_copy(
            rows_v.at[pl.ds(half * _GROWS, _GROWS)],
            out_hbm.at[pl.ds(base + g * _GROWS, _GROWS)],
            wsem.at[half],
        )

    def drain_writeback(half):
        pltpu.make_async_copy(
            table_hbm.at[pl.ds(0, _GROWS)],
            rows_v.at[pl.ds(half * _GROWS, _GROWS)],
            wsem.at[half],
        ).wait()

    fire_group(0, 0)

    def step(g, carry):
        h = lax.rem(g, 2)
        hn = 1 - h

        @pl.when(g + 1 < _NGROUP)
        def _():
            @pl.when(g >= 1)
            def _():
                drain_writeback(hn)

            fire_group(g + 1, hn)

        drain_gathers(h)
        fire_writeback(g, h)
        return carry

    lax.fori_loop(0, _NGROUP, step, 0)

    drain_writeback((_NGROUP - 1) % 2)
    drain_writeback(_NGROUP % 2)


@jax.jit
def _gather(idx, table):
    mesh = plsc.VectorSubcoreMesh(core_axis_name="c", subcore_axis_name="s")
    return pl.kernel(
        _gather_body,
        out_type=jax.ShapeDtypeStruct((_B, _EMBED_DIM), jnp.float32),
        mesh=mesh,
        scratch_types=[
            pltpu.VMEM((_NCHUNK, _CHUNK), jnp.int32),
            pltpu.VMEM((2 * _GROWS, _EMBED_DIM), jnp.float32),
            pltpu.SemaphoreType.DMA((2,)),
            pltpu.SemaphoreType.DMA((2,)),
        ],
        compiler_params=pltpu.CompilerParams(use_tc_tiling_on_sc=False),
    )(idx, table)


def kernel(edge_index, embedding_weight):
    idx = edge_index.reshape(_NW, _NCHUNK, _CHUNK).astype(jnp.int32)
    out = _gather(idx, embedding_weight)
    return out.reshape(2, _NUM_EDGES, _EMBED_DIM)
